# baseline (speedup 1.0000x reference)
import jax
import jax.numpy as jnp
from jax.experimental import pallas as pl
from jax.experimental.pallas import tpu as pltpu
from jax.experimental.pallas import tpu_sc as plsc

_K = 8192
_C = 256
_BN = 512
_BK = 8192
_GW = 128


_BIAS = 1.5
_EXP_BITS = 0x3F800000


def _argmin_body(zsq_ref, z2b_ref, emb_ref, esqb_ref, idx_ref, loss_ref,
                 minkey_scr):
    i = pl.program_id(0)
    j = pl.program_id(1)
    nb_k = pl.num_programs(1)

    emb_b = emb_ref[...].astype(jnp.bfloat16)
    p2 = jax.lax.dot_general(
        emb_b, z2b_ref[...], (((1,), (1,)), ((), ())),
        preferred_element_type=jnp.float32)
    key = esqb_ref[...] - p2
    kbits = jax.lax.bitcast_convert_type(key, jnp.int32)
    iota = jax.lax.broadcasted_iota(jnp.int32, key.shape, 0)
    packed = ((kbits << 8) & jnp.int32(0x7FFFE000)) | iota
    m = jnp.min(packed, axis=0, keepdims=True)

    @pl.when(j == 0)
    def _():
        minkey_scr[...] = m

    @pl.when(j > 0)
    def _():
        prev = minkey_scr[...]
        cur = m | jnp.int32(j * _BK)
        minkey_scr[...] = jnp.where(cur < prev, cur, prev)

    @pl.when(j == nb_k - 1)
    def _():
        final = minkey_scr[...]
        idx_ref[...] = final & jnp.int32(_K - 1)
        vbits = ((final >> 13) << 5) | jnp.int32(_EXP_BITS)
        mval = jax.lax.bitcast_convert_type(vbits, jnp.float32) - _BIAS
        s = jnp.sum(zsq_ref[...] + mval).reshape(1, 1)

        @pl.when(i == 0)
        def _():
            loss_ref[...] = s

        @pl.when(i > 0)
        def _():
            loss_ref[...] += s


def _distance_argmin(z2b, zsq_row, emb, esq_col):
    n = z2b.shape[0]
    grid = (n // _BN, _K // _BK)
    return pl.pallas_call(
        _argmin_body,
        grid=grid,
        in_specs=[
            pl.BlockSpec((1, _BN), lambda i, j: (0, i)),
            pl.BlockSpec((_BN, _C), lambda i, j: (i, 0)),
            pl.BlockSpec((_BK, _C), lambda i, j: (j, 0)),
            pl.BlockSpec((_BK, 1), lambda i, j: (j, 0)),
        ],
        out_specs=[
            pl.BlockSpec((1, _BN), lambda i, j: (0, i)),
            pl.BlockSpec((1, 1), lambda i, j: (0, 0)),
        ],
        out_shape=[
            jax.ShapeDtypeStruct((1, n), jnp.int32),
            jax.ShapeDtypeStruct((1, 1), jnp.float32),
        ],
        scratch_shapes=[
            pltpu.VMEM((1, _BN), jnp.int32),
        ],
        compiler_params=pltpu.CompilerParams(
            dimension_semantics=("arbitrary", "arbitrary")),
    )(zsq_row, z2b, emb, esq_col)


def _gather_rows(embeddings, idx_row):
    n = idx_row.shape[1]

    @pl.kernel(
        out_type=jax.ShapeDtypeStruct((n, _C), jnp.float32),
        mesh=plsc.VectorSubcoreMesh(core_axis_name="c", subcore_axis_name="s"))
    def gather_kernel(emb_hbm, i_hbm, o_hbm):
        def body(i_vmem, o_vmem):
            pltpu.sync_copy(emb_hbm.at[i_vmem.at[0]], o_vmem)

        pltpu.emit_pipeline(
            body,
            grid=(n // _GW,),
            in_specs=[pl.BlockSpec((1, _GW), lambda i: (0, i))],
            out_specs=[pl.BlockSpec((_GW, _C), lambda i: (i, 0))],
            core_axis_name=("c", "s"),
            dimension_semantics=(pltpu.PARALLEL,),
        )(i_hbm, o_hbm)

    return gather_kernel(embeddings, idx_row)


def kernel(z, embeddings):
    b, c, h, w = z.shape
    n = b * h * w
    flat_z = jnp.transpose(z, (0, 2, 3, 1)).reshape(-1, c)
    z2b = (2.0 * flat_z).astype(jnp.bfloat16)
    zsq_row = jnp.sum(z ** 2, axis=1).reshape(1, -1)
    esqb_col = (jnp.sum(embeddings ** 2, axis=1)
                + jnp.float32(_BIAS)).reshape(-1, 1)

    idx, loss_sum = _distance_argmin(z2b, zsq_row, embeddings, esqb_col)

    quant_rows = _gather_rows(embeddings, idx)
    quantized = jnp.transpose(quant_rows.reshape(b, h, w, c), (0, 3, 1, 2))

    loss = (1.0 + 0.25) * loss_sum[0, 0] / jnp.float32(n * c)
    return (quantized, loss)


# speedup vs baseline: 1.0557x; 1.0557x over previous
import jax
import jax.numpy as jnp
from jax.experimental import pallas as pl
from jax.experimental.pallas import tpu as pltpu
from jax.experimental.pallas import tpu_sc as plsc

_K = 8192
_C = 256
_BN = 1024
_BK = 8192
_GW = 128


_BIAS = 1.5
_EXP_BITS = 0x3F800000


def _argmin_body(zsq_ref, z2b_ref, emb_ref, esqb_ref, idx_ref, loss_ref,
                 minkey_scr):
    i = pl.program_id(0)
    j = pl.program_id(1)
    nb_k = pl.num_programs(1)

    emb_b = emb_ref[...].astype(jnp.bfloat16)
    p2 = jax.lax.dot_general(
        emb_b, z2b_ref[...], (((1,), (1,)), ((), ())),
        preferred_element_type=jnp.float32)
    key = esqb_ref[...] - p2
    kbits = jax.lax.bitcast_convert_type(key, jnp.int32)
    iota = jax.lax.broadcasted_iota(jnp.int32, key.shape, 0)
    packed = ((kbits << 8) & jnp.int32(0x7FFFE000)) | iota
    m = jnp.min(packed, axis=0, keepdims=True)

    @pl.when(j == 0)
    def _():
        minkey_scr[...] = m

    @pl.when(j > 0)
    def _():
        prev = minkey_scr[...]
        cur = m | jnp.int32(j * _BK)
        minkey_scr[...] = jnp.where(cur < prev, cur, prev)

    @pl.when(j == nb_k - 1)
    def _():
        final = minkey_scr[...]
        idx_ref[...] = final & jnp.int32(_K - 1)
        vbits = ((final >> 13) << 5) | jnp.int32(_EXP_BITS)
        mval = jax.lax.bitcast_convert_type(vbits, jnp.float32) - _BIAS
        s = jnp.sum(zsq_ref[...] + mval).reshape(1, 1)

        @pl.when(i == 0)
        def _():
            loss_ref[...] = s

        @pl.when(i > 0)
        def _():
            loss_ref[...] += s


def _distance_argmin(z2b, zsq_row, emb, esq_col):
    n = z2b.shape[0]
    grid = (n // _BN, _K // _BK)
    return pl.pallas_call(
        _argmin_body,
        grid=grid,
        in_specs=[
            pl.BlockSpec((1, _BN), lambda i, j: (0, i)),
            pl.BlockSpec((_BN, _C), lambda i, j: (i, 0)),
            pl.BlockSpec((_BK, _C), lambda i, j: (j, 0)),
            pl.BlockSpec((_BK, 1), lambda i, j: (j, 0)),
        ],
        out_specs=[
            pl.BlockSpec((1, _BN), lambda i, j: (0, i)),
            pl.BlockSpec((1, 1), lambda i, j: (0, 0)),
        ],
        out_shape=[
            jax.ShapeDtypeStruct((1, n), jnp.int32),
            jax.ShapeDtypeStruct((1, 1), jnp.float32),
        ],
        scratch_shapes=[
            pltpu.VMEM((1, _BN), jnp.int32),
        ],
        compiler_params=pltpu.CompilerParams(
            dimension_semantics=("arbitrary", "arbitrary")),
    )(zsq_row, z2b, emb, esq_col)


def _gather_rows(embeddings, idx_row):
    n = idx_row.shape[1]

    @pl.kernel(
        out_type=jax.ShapeDtypeStruct((n, _C), jnp.float32),
        mesh=plsc.VectorSubcoreMesh(core_axis_name="c", subcore_axis_name="s"))
    def gather_kernel(emb_hbm, i_hbm, o_hbm):
        def body(i_vmem, o_vmem):
            pltpu.sync_copy(emb_hbm.at[i_vmem.at[0]], o_vmem)

        pltpu.emit_pipeline(
            body,
            grid=(n // _GW,),
            in_specs=[pl.BlockSpec((1, _GW), lambda i: (0, i))],
            out_specs=[pl.BlockSpec((_GW, _C), lambda i: (i, 0))],
            core_axis_name=("c", "s"),
            dimension_semantics=(pltpu.PARALLEL,),
        )(i_hbm, o_hbm)

    return gather_kernel(embeddings, idx_row)


def kernel(z, embeddings):
    b, c, h, w = z.shape
    n = b * h * w
    flat_z = jnp.transpose(z, (0, 2, 3, 1)).reshape(-1, c)
    z2b = (2.0 * flat_z).astype(jnp.bfloat16)
    zsq_row = jnp.sum(z ** 2, axis=1).reshape(1, -1)
    esqb_col = (jnp.sum(embeddings ** 2, axis=1)
                + jnp.float32(_BIAS)).reshape(-1, 1)

    idx, loss_sum = _distance_argmin(z2b, zsq_row, embeddings, esqb_col)

    quant_rows = _gather_rows(embeddings, idx)
    quantized = jnp.transpose(quant_rows.reshape(b, h, w, c), (0, 3, 1, 2))

    loss = (1.0 + 0.25) * loss_sum[0, 0] / jnp.float32(n * c)
    return (quantized, loss)


# speedup vs baseline: 1.0786x; 1.0217x over previous
import jax
import jax.numpy as jnp
from jax.experimental import pallas as pl
from jax.experimental.pallas import tpu as pltpu
from jax.experimental.pallas import tpu_sc as plsc

_K = 8192
_C = 256
_BN = 1024
_BK = 8192
_GW = 128


_BIAS = 1.5
_EXP_BITS = 0x3F800000


def _argmin_body(zsq_ref, z2b_ref, emb_ref, esqb_ref, idx_ref, loss_ref,
                 minkey_scr):
    i = pl.program_id(0)
    j = pl.program_id(1)
    nb_k = pl.num_programs(1)

    p2 = jax.lax.dot_general(
        emb_ref[...], z2b_ref[...], (((1,), (1,)), ((), ())),
        preferred_element_type=jnp.float32)
    key = esqb_ref[...] - p2
    kbits = jax.lax.bitcast_convert_type(key, jnp.int32)
    iota = jax.lax.broadcasted_iota(jnp.int32, key.shape, 0)
    packed = ((kbits << 8) & jnp.int32(0x7FFFE000)) | iota
    m = jnp.min(packed, axis=0, keepdims=True)

    @pl.when(j == 0)
    def _():
        minkey_scr[...] = m

    @pl.when(j > 0)
    def _():
        prev = minkey_scr[...]
        cur = m | jnp.int32(j * _BK)
        minkey_scr[...] = jnp.where(cur < prev, cur, prev)

    @pl.when(j == nb_k - 1)
    def _():
        final = minkey_scr[...]
        idx_ref[...] = final & jnp.int32(_K - 1)
        vbits = ((final >> 13) << 5) | jnp.int32(_EXP_BITS)
        mval = jax.lax.bitcast_convert_type(vbits, jnp.float32) - _BIAS
        s = jnp.sum(zsq_ref[...] + mval).reshape(1, 1)

        @pl.when(i == 0)
        def _():
            loss_ref[...] = s

        @pl.when(i > 0)
        def _():
            loss_ref[...] += s


def _distance_argmin(z2b, zsq_row, emb, esq_col):
    n = z2b.shape[0]
    grid = (n // _BN, _K // _BK)
    return pl.pallas_call(
        _argmin_body,
        grid=grid,
        in_specs=[
            pl.BlockSpec((1, _BN), lambda i, j: (0, i)),
            pl.BlockSpec((_BN, _C), lambda i, j: (i, 0)),
            pl.BlockSpec((_BK, _C), lambda i, j: (j, 0)),
            pl.BlockSpec((_BK, 1), lambda i, j: (j, 0)),
        ],
        out_specs=[
            pl.BlockSpec((1, _BN), lambda i, j: (0, i)),
            pl.BlockSpec((1, 1), lambda i, j: (0, 0)),
        ],
        out_shape=[
            jax.ShapeDtypeStruct((1, n), jnp.int32),
            jax.ShapeDtypeStruct((1, 1), jnp.float32),
        ],
        scratch_shapes=[
            pltpu.VMEM((1, _BN), jnp.int32),
        ],
        compiler_params=pltpu.CompilerParams(
            dimension_semantics=("arbitrary", "arbitrary")),
    )(zsq_row, z2b, emb, esq_col)


def _gather_rows(embeddings, idx_row):
    n = idx_row.shape[1]

    @pl.kernel(
        out_type=jax.ShapeDtypeStruct((n, _C), jnp.float32),
        mesh=plsc.VectorSubcoreMesh(core_axis_name="c", subcore_axis_name="s"))
    def gather_kernel(emb_hbm, i_hbm, o_hbm):
        def body(i_vmem, o_vmem):
            pltpu.sync_copy(emb_hbm.at[i_vmem.at[0]], o_vmem)

        pltpu.emit_pipeline(
            body,
            grid=(n // _GW,),
            in_specs=[pl.BlockSpec((1, _GW), lambda i: (0, i))],
            out_specs=[pl.BlockSpec((_GW, _C), lambda i: (i, 0))],
            core_axis_name=("c", "s"),
            dimension_semantics=(pltpu.PARALLEL,),
        )(i_hbm, o_hbm)

    return gather_kernel(embeddings, idx_row)


def kernel(z, embeddings):
    b, c, h, w = z.shape
    n = b * h * w
    flat_z = jnp.transpose(z, (0, 2, 3, 1)).reshape(-1, c)
    z2b = (2.0 * flat_z).astype(jnp.bfloat16)
    zsq_row = jnp.sum(z ** 2, axis=1).reshape(1, -1)
    esqb_col = (jnp.sum(embeddings ** 2, axis=1)
                + jnp.float32(_BIAS)).reshape(-1, 1)

    emb_b = embeddings.astype(jnp.bfloat16)
    idx, loss_sum = _distance_argmin(z2b, zsq_row, emb_b, esqb_col)

    quant_rows = _gather_rows(embeddings, idx)
    quantized = jnp.transpose(quant_rows.reshape(b, h, w, c), (0, 3, 1, 2))

    loss = (1.0 + 0.25) * loss_sum[0, 0] / jnp.float32(n * c)
    return (quantized, loss)
